# Initial kernel scaffold; baseline (speedup 1.0000x reference)
import jax
import jax.numpy as jnp
from jax.experimental import pallas as pl


def kernel(x, W, b):
    raise NotImplementedError("write your pallas kernel here")


# speedup vs baseline: 255.3346x; 255.3346x over previous
import functools

import jax
import jax.numpy as jnp
from jax import lax
from jax.experimental import pallas as pl
from jax.experimental.pallas import tpu as pltpu

TAG = 32
_SIGN = -(2 ** 31)


def _tc_body(x_ref, wt_ref, b_ref, o_ref):
    x = x_ref[...]
    xc = x - jnp.mean(x, axis=1, keepdims=True)
    kc = jnp.dot(xc, wt_ref[...], preferred_element_type=jnp.float32)
    kc = kc + b_ref[...]

    b = lax.bitcast_convert_type(kc, jnp.int32)
    s = jnp.where(b >= 0, b, b ^ jnp.int32(0x7FFFFFFF))

    rows = kc.shape[0]

    def step(i, cur):
        bit = lax.shift_left(jnp.int32(1), jnp.int32(31) - i)
        cand_u = cur | bit
        cand_s = cand_u ^ jnp.int32(_SIGN)
        cnt = jnp.sum((s >= cand_s).astype(jnp.int32), axis=1, keepdims=True)
        return jnp.where(cnt >= TAG, cand_u, cur)

    cur = lax.fori_loop(0, 32, step, jnp.zeros((rows, 1), jnp.int32))
    t_s = cur ^ jnp.int32(_SIGN)
    o_ref[...] = jnp.where(s >= t_s, kc, 0.0)


@jax.jit
def kernel(x, W, b):
    batch, in_dim = x.shape
    out_dim = W.shape[0]
    wt = W.T
    b2 = b.reshape(1, out_dim)
    br = min(512, batch)
    grid = (batch // br,)
    return pl.pallas_call(
        _tc_body,
        grid=grid,
        in_specs=[
            pl.BlockSpec((br, in_dim), lambda i: (i, 0)),
            pl.BlockSpec((in_dim, out_dim), lambda i: (0, 0)),
            pl.BlockSpec((1, out_dim), lambda i: (0, 0)),
        ],
        out_specs=pl.BlockSpec((br, out_dim), lambda i: (i, 0)),
        out_shape=jax.ShapeDtypeStruct((batch, out_dim), jnp.float32),
        compiler_params=pltpu.CompilerParams(
            dimension_semantics=("parallel",)
        ),
    )(x, wt, b2)
